# baseline (speedup 1.0000x reference)
import functools

import jax
import jax.numpy as jnp
from jax import lax
from jax.experimental import pallas as pl
from jax.experimental.pallas import tpu as pltpu
from jax.experimental.pallas import tpu_sc as plsc

P = 16
V = 16
D = 16
N_CONT = 262144
N_CATEG = 262144
NDAYS = 3664

NC = 2
NS = 16
ROWS_T = N_CONT // NS
CHUNK = 2048
NCHUNK = ROWS_T // CHUNK
MB = CHUNK // 16
CSTRIDE = CHUNK + 8

TWOPI = 6.283185307179586
INV2PI = 1.0 / TWOPI

SIN_C = (0.9999791148943297, -0.1666240153829831, 0.00830884993122673,
         -0.00019263169952744158, 2.147049615625063e-06)
COS_C = (0.9999992107412203, -0.4999942131500665, 0.04165977758594538,
         -0.0013858789204833017, 2.4202932054760706e-05,
         -2.1972921876445284e-07)

INV17 = tuple(10000.0 ** (-(2 * j) / 17.0) for j in range(9))
INV16 = tuple(10000.0 ** (-(2 * j) / 16.0) for j in range(8))


def _range_reduce(ang):
    q = (ang * INV2PI + 0.5).astype(jnp.int32).astype(jnp.float32)
    return ang - q * TWOPI


def _sin_poly(r, r2):
    s = jnp.float32(SIN_C[-1])
    for c in SIN_C[-2::-1]:
        s = s * r2 + c
    return s * r


def _cos_poly(r2):
    c = jnp.float32(COS_C[-1])
    for cc in COS_C[-2::-1]:
        c = c * r2 + cc
    return c


def _embed_body(pc_hbm, vals_hbm, dc_hbm, pg_hbm, vg_hbm, dg_hbm,
                pair_hbm, valtab_hbm, outc_hbm, outg_hbm,
                tab_v, pair_v, pair_p, valtab_v, comb_v,
                b_p, b_d, b_vi, b_vf, out_v, sem):
    cid = lax.axis_index("c")
    sid = lax.axis_index("s")
    base = sid * ROWS_T
    iota = lax.iota(jnp.int32, 16)

    pltpu.sync_copy(pair_hbm, pair_v)

    def repack(i, c0):
        pair_p[pl.ds(i * 17, 16)] = pair_v[pl.ds(i * 16, 16)]
        return c0

    lax.fori_loop(0, P, repack, 0)

    def _build_tab(inv, with_col16):
        def build(t, c0):
            d_f = (iota + t * 16).astype(jnp.float32)
            db = (iota + t * 16) * 17
            for j in range(8):
                r = _range_reduce(d_f * inv[j])
                r2 = r * r
                plsc.store_scatter(tab_v, [db + (2 * j)], _sin_poly(r, r2))
                plsc.store_scatter(tab_v, [db + (2 * j + 1)], _cos_poly(r2))
            if with_col16:
                r = _range_reduce(d_f * INV17[8])
                plsc.store_scatter(tab_v, [db + 16], _sin_poly(r, r * r))
            return c0

        lax.fori_loop(0, NDAYS // 16, build, 0)

    @pl.when(cid == 0)
    def _cont_side():
        _build_tab(INV17, True)

        def chunk_body(c, carry):
            off = base + c * CHUNK
            pltpu.sync_copy(pc_hbm.at[pl.ds(off, CHUNK)], b_p)
            pltpu.sync_copy(dc_hbm.at[pl.ds(off, CHUNK)], b_d)
            pltpu.sync_copy(vals_hbm.at[pl.ds(off, CHUNK)], b_vf)

            def mb_body(m, c2):
                sl = pl.ds(m * 16, 16)
                db = b_d[sl] * 17
                pb = b_p[sl] * 17
                for k in range(16):
                    pe = plsc.load_gather(tab_v, [db + k])
                    pr = plsc.load_gather(pair_p, [pb + k])
                    out_v[pl.ds(k * CSTRIDE + m * 16, 16)] = pe + pr
                pe16 = plsc.load_gather(tab_v, [db + 16])
                out_v[pl.ds(16 * CSTRIDE + m * 16, 16)] = pe16 + b_vf[sl]
                return c2

            lax.fori_loop(0, MB, mb_body, 0)
            cps = [pltpu.async_copy(out_v.at[pl.ds(k * CSTRIDE, CHUNK)],
                                    outc_hbm.at[pl.ds(k * N_CONT + off, CHUNK)],
                                    sem)
                   for k in range(17)]
            for cp in cps:
                cp.wait()
            return carry

        lax.fori_loop(0, NCHUNK, chunk_body, 0)

    @pl.when(cid == 1)
    def _categ_side():
        pltpu.sync_copy(valtab_hbm, valtab_v)

        def build_comb(i, c0):
            pr = pair_v[pl.ds((i >> 4) * 16, 16)]
            comb_v[pl.ds(i * 17, 16)] = pr + valtab_v[pl.ds(i * 16, 16)]
            return c0

        lax.fori_loop(0, P * V, build_comb, 0)

        _build_tab(INV16, False)

        def chunk_body(c, carry):
            off = base + c * CHUNK
            pltpu.sync_copy(pg_hbm.at[pl.ds(off, CHUNK)], b_p)
            pltpu.sync_copy(dg_hbm.at[pl.ds(off, CHUNK)], b_d)
            pltpu.sync_copy(vg_hbm.at[pl.ds(off, CHUNK)], b_vi)

            def mb_body(m, c2):
                sl = pl.ds(m * 16, 16)
                db = b_d[sl] * 17
                cb = (b_p[sl] * 16 + b_vi[sl]) * 17
                for k in range(16):
                    pe = plsc.load_gather(tab_v, [db + k])
                    cm = plsc.load_gather(comb_v, [cb + k])
                    out_v[pl.ds(k * CSTRIDE + m * 16, 16)] = pe + cm
                return c2

            lax.fori_loop(0, MB, mb_body, 0)
            cps = [pltpu.async_copy(out_v.at[pl.ds(k * CSTRIDE, CHUNK)],
                                    outg_hbm.at[pl.ds(k * N_CATEG + off, CHUNK)],
                                    sem)
                   for k in range(16)]
            for cp in cps:
                cp.wait()
            return carry

        lax.fori_loop(0, NCHUNK, chunk_body, 0)


@jax.jit
def kernel(ent_attr_ids_cont, vals_cont, days_cont,
           ent_attr_ids_categ, vocab_ids_categ, days_categ,
           pair_emb, categ_val_emb):
    mesh = plsc.VectorSubcoreMesh(core_axis_name="c", subcore_axis_name="s")
    f = pl.kernel(
        _embed_body,
        out_type=(jax.ShapeDtypeStruct((17 * N_CONT,), jnp.float32),
                  jax.ShapeDtypeStruct((16 * N_CATEG,), jnp.float32)),
        mesh=mesh,
        compiler_params=pltpu.CompilerParams(needs_layout_passes=False),
        scratch_types=[
            pltpu.VMEM((NDAYS * 17,), jnp.float32),
            pltpu.VMEM((P * D,), jnp.float32),
            pltpu.VMEM((P * 17,), jnp.float32),
            pltpu.VMEM((P * V * D,), jnp.float32),
            pltpu.VMEM((P * V * 17,), jnp.float32),
            pltpu.VMEM((CHUNK,), jnp.int32),
            pltpu.VMEM((CHUNK,), jnp.int32),
            pltpu.VMEM((CHUNK,), jnp.int32),
            pltpu.VMEM((CHUNK,), jnp.float32),
            pltpu.VMEM((17 * CSTRIDE,), jnp.float32),
            pltpu.SemaphoreType.DMA,
        ],
    )
    outc, outg = f(ent_attr_ids_cont.astype(jnp.int32), vals_cont,
                   days_cont.astype(jnp.int32),
                   ent_attr_ids_categ.astype(jnp.int32),
                   vocab_ids_categ.astype(jnp.int32),
                   days_categ.astype(jnp.int32),
                   pair_emb.reshape(P * D), categ_val_emb.reshape(P * V * D))
    return outc.reshape(17, N_CONT).T, outg.reshape(16, N_CATEG).T


# speedup vs baseline: 1.2067x; 1.2067x over previous
import functools

import jax
import jax.numpy as jnp
from jax import lax
from jax.experimental import pallas as pl
from jax.experimental.pallas import tpu as pltpu
from jax.experimental.pallas import tpu_sc as plsc

P = 16
V = 16
D = 16
N_CONT = 262144
N_CATEG = 262144
NDAYS = 3664

NC = 2
NS = 16
ROWS_T = N_CONT // NS
CHUNK = 2048
NCHUNK = ROWS_T // CHUNK
MB = CHUNK // 16
CSTRIDE = CHUNK + 8

TWOPI = 6.283185307179586
INV2PI = 1.0 / TWOPI

SIN_C = (0.9999791148943297, -0.1666240153829831, 0.00830884993122673,
         -0.00019263169952744158, 2.147049615625063e-06)
COS_C = (0.9999992107412203, -0.4999942131500665, 0.04165977758594538,
         -0.0013858789204833017, 2.4202932054760706e-05,
         -2.1972921876445284e-07)

INV17 = tuple(10000.0 ** (-(2 * j) / 17.0) for j in range(9))
INV16 = tuple(10000.0 ** (-(2 * j) / 16.0) for j in range(8))


def _range_reduce(ang):
    q = (ang * INV2PI + 0.5).astype(jnp.int32).astype(jnp.float32)
    return ang - q * TWOPI


def _sin_poly(r, r2):
    s = jnp.float32(SIN_C[-1])
    for c in SIN_C[-2::-1]:
        s = s * r2 + c
    return s * r


def _cos_poly(r2):
    c = jnp.float32(COS_C[-1])
    for cc in COS_C[-2::-1]:
        c = c * r2 + cc
    return c


def _embed_body(pc_hbm, vals_hbm, dc_hbm, pg_hbm, vg_hbm, dg_hbm,
                pair_hbm, valtab_hbm, outc_hbm, outg_hbm,
                tab_v, pair_v, pair_p, valtab_v, comb_v,
                b_p, b_d, b_vi, b_vf, out_v, sem):
    cid = lax.axis_index("c")
    sid = lax.axis_index("s")
    base = sid * ROWS_T
    iota = lax.iota(jnp.int32, 16)

    pltpu.sync_copy(pair_hbm, pair_v)

    def repack(i, c0):
        pair_p[pl.ds(i * 17, 16)] = pair_v[pl.ds(i * 16, 16)]
        return c0

    lax.fori_loop(0, P, repack, 0)

    def _build_tab(inv, with_col16):
        @plsc.parallel_loop(0, NDAYS // 16, step=1, unroll=2)
        def build(t):
            d_f = (iota + t * 16).astype(jnp.float32)
            db = (iota + t * 16) * 17
            for j in range(8):
                r = _range_reduce(d_f * inv[j])
                r2 = r * r
                plsc.store_scatter(tab_v, [db + (2 * j)], _sin_poly(r, r2))
                plsc.store_scatter(tab_v, [db + (2 * j + 1)], _cos_poly(r2))
            if with_col16:
                r = _range_reduce(d_f * INV17[8])
                plsc.store_scatter(tab_v, [db + 16], _sin_poly(r, r * r))

    @pl.when(cid == 0)
    def _cont_side():
        _build_tab(INV17, True)

        def chunk_body(c, carry):
            off = base + c * CHUNK
            pltpu.sync_copy(pc_hbm.at[pl.ds(off, CHUNK)], b_p)
            pltpu.sync_copy(dc_hbm.at[pl.ds(off, CHUNK)], b_d)
            pltpu.sync_copy(vals_hbm.at[pl.ds(off, CHUNK)], b_vf)

            @plsc.parallel_loop(0, MB, step=1, unroll=4)
            def mb_body(m):
                sl = pl.ds(m * 16, 16)
                db = b_d[sl] * 17
                pb = b_p[sl] * 17
                for k in range(16):
                    pe = plsc.load_gather(tab_v, [db + k])
                    pr = plsc.load_gather(pair_p, [pb + k])
                    out_v[pl.ds(k * CSTRIDE + m * 16, 16)] = pe + pr
                pe16 = plsc.load_gather(tab_v, [db + 16])
                out_v[pl.ds(16 * CSTRIDE + m * 16, 16)] = pe16 + b_vf[sl]
            cps = [pltpu.async_copy(out_v.at[pl.ds(k * CSTRIDE, CHUNK)],
                                    outc_hbm.at[pl.ds(k * N_CONT + off, CHUNK)],
                                    sem)
                   for k in range(17)]
            for cp in cps:
                cp.wait()
            return carry

        lax.fori_loop(0, NCHUNK, chunk_body, 0)

    @pl.when(cid == 1)
    def _categ_side():
        pltpu.sync_copy(valtab_hbm, valtab_v)

        def build_comb(i, c0):
            pr = pair_v[pl.ds((i >> 4) * 16, 16)]
            comb_v[pl.ds(i * 17, 16)] = pr + valtab_v[pl.ds(i * 16, 16)]
            return c0

        lax.fori_loop(0, P * V, build_comb, 0)

        _build_tab(INV16, False)

        def chunk_body(c, carry):
            off = base + c * CHUNK
            pltpu.sync_copy(pg_hbm.at[pl.ds(off, CHUNK)], b_p)
            pltpu.sync_copy(dg_hbm.at[pl.ds(off, CHUNK)], b_d)
            pltpu.sync_copy(vg_hbm.at[pl.ds(off, CHUNK)], b_vi)

            @plsc.parallel_loop(0, MB, step=1, unroll=4)
            def mb_body(m):
                sl = pl.ds(m * 16, 16)
                db = b_d[sl] * 17
                cb = (b_p[sl] * 16 + b_vi[sl]) * 17
                for k in range(16):
                    pe = plsc.load_gather(tab_v, [db + k])
                    cm = plsc.load_gather(comb_v, [cb + k])
                    out_v[pl.ds(k * CSTRIDE + m * 16, 16)] = pe + cm
            cps = [pltpu.async_copy(out_v.at[pl.ds(k * CSTRIDE, CHUNK)],
                                    outg_hbm.at[pl.ds(k * N_CATEG + off, CHUNK)],
                                    sem)
                   for k in range(16)]
            for cp in cps:
                cp.wait()
            return carry

        lax.fori_loop(0, NCHUNK, chunk_body, 0)


@jax.jit
def kernel(ent_attr_ids_cont, vals_cont, days_cont,
           ent_attr_ids_categ, vocab_ids_categ, days_categ,
           pair_emb, categ_val_emb):
    mesh = plsc.VectorSubcoreMesh(core_axis_name="c", subcore_axis_name="s")
    f = pl.kernel(
        _embed_body,
        out_type=(jax.ShapeDtypeStruct((17 * N_CONT,), jnp.float32),
                  jax.ShapeDtypeStruct((16 * N_CATEG,), jnp.float32)),
        mesh=mesh,
        compiler_params=pltpu.CompilerParams(needs_layout_passes=False),
        scratch_types=[
            pltpu.VMEM((NDAYS * 17,), jnp.float32),
            pltpu.VMEM((P * D,), jnp.float32),
            pltpu.VMEM((P * 17,), jnp.float32),
            pltpu.VMEM((P * V * D,), jnp.float32),
            pltpu.VMEM((P * V * 17,), jnp.float32),
            pltpu.VMEM((CHUNK,), jnp.int32),
            pltpu.VMEM((CHUNK,), jnp.int32),
            pltpu.VMEM((CHUNK,), jnp.int32),
            pltpu.VMEM((CHUNK,), jnp.float32),
            pltpu.VMEM((17 * CSTRIDE,), jnp.float32),
            pltpu.SemaphoreType.DMA,
        ],
    )
    outc, outg = f(ent_attr_ids_cont.astype(jnp.int32), vals_cont,
                   days_cont.astype(jnp.int32),
                   ent_attr_ids_categ.astype(jnp.int32),
                   vocab_ids_categ.astype(jnp.int32),
                   days_categ.astype(jnp.int32),
                   pair_emb.reshape(P * D), categ_val_emb.reshape(P * V * D))
    return outc.reshape(17, N_CONT).T, outg.reshape(16, N_CATEG).T


# speedup vs baseline: 1.2517x; 1.0373x over previous
import functools

import jax
import jax.numpy as jnp
from jax import lax
from jax.experimental import pallas as pl
from jax.experimental.pallas import tpu as pltpu
from jax.experimental.pallas import tpu_sc as plsc

P = 16
V = 16
D = 16
N_CONT = 262144
N_CATEG = 262144
NDAYS = 3664

NC = 2
NS = 16
ROWS_T = N_CONT // NS
CHUNK = 1024
NCHUNK = ROWS_T // CHUNK
MB = CHUNK // 16
CSTRIDE = CHUNK + 8

TWOPI = 6.283185307179586
INV2PI = 1.0 / TWOPI

SIN_C = (0.9999791148943297, -0.1666240153829831, 0.00830884993122673,
         -0.00019263169952744158, 2.147049615625063e-06)
COS_C = (0.9999992107412203, -0.4999942131500665, 0.04165977758594538,
         -0.0013858789204833017, 2.4202932054760706e-05,
         -2.1972921876445284e-07)

INV17 = tuple(10000.0 ** (-(2 * j) / 17.0) for j in range(9))
INV16 = tuple(10000.0 ** (-(2 * j) / 16.0) for j in range(8))


def _range_reduce(ang):
    q = (ang * INV2PI + 0.5).astype(jnp.int32).astype(jnp.float32)
    return ang - q * TWOPI


def _sin_poly(r, r2):
    s = jnp.float32(SIN_C[-1])
    for c in SIN_C[-2::-1]:
        s = s * r2 + c
    return s * r


def _cos_poly(r2):
    c = jnp.float32(COS_C[-1])
    for cc in COS_C[-2::-1]:
        c = c * r2 + cc
    return c


def _embed_body(pc_hbm, vals_hbm, dc_hbm, pg_hbm, vg_hbm, dg_hbm,
                pair_hbm, valtab_hbm, outc_hbm, outg_hbm,
                tab_v, pair_v, pair_p, valtab_v, comb_v,
                b_p, b_d, b_vi, b_vf, out_a, out_b, sem_a, sem_b):
    cid = lax.axis_index("c")
    sid = lax.axis_index("s")
    base = sid * ROWS_T
    iota = lax.iota(jnp.int32, 16)

    pltpu.sync_copy(pair_hbm, pair_v)

    def repack(i, c0):
        pair_p[pl.ds(i * 17, 16)] = pair_v[pl.ds(i * 16, 16)]
        return c0

    lax.fori_loop(0, P, repack, 0)

    def _build_tab(inv, with_col16):
        @plsc.parallel_loop(0, NDAYS // 16, step=1, unroll=2)
        def build(t):
            d_f = (iota + t * 16).astype(jnp.float32)
            db = (iota + t * 16) * 17
            for j in range(8):
                r = _range_reduce(d_f * inv[j])
                r2 = r * r
                plsc.store_scatter(tab_v, [db + (2 * j)], _sin_poly(r, r2))
                plsc.store_scatter(tab_v, [db + (2 * j + 1)], _cos_poly(r2))
            if with_col16:
                r = _range_reduce(d_f * INV17[8])
                plsc.store_scatter(tab_v, [db + 16], _sin_poly(r, r * r))

    def _run_chunks(compute_chunk, out_hbm, ncols, n_total):
        def issue_out(c, buf, sem):
            off = base + c * CHUNK
            for k in range(ncols):
                pltpu.async_copy(buf.at[pl.ds(k * CSTRIDE, CHUNK)],
                                 out_hbm.at[pl.ds(k * n_total + off, CHUNK)],
                                 sem)

        def wait_out(c, buf, sem):
            off = base + jnp.maximum(c, 0) * CHUNK
            for k in range(ncols):
                pltpu.make_async_copy(
                    buf.at[pl.ds(k * CSTRIDE, CHUNK)],
                    out_hbm.at[pl.ds(k * n_total + off, CHUNK)],
                    sem).wait()

        def pair_body(cc, carry):
            c0 = cc * 2
            c1 = c0 + 1

            @pl.when(cc > 0)
            def _wa():
                wait_out(c0 - 2, out_a, sem_a)

            compute_chunk(c0, out_a)
            issue_out(c0, out_a, sem_a)

            @pl.when(cc > 0)
            def _wb():
                wait_out(c1 - 2, out_b, sem_b)

            compute_chunk(c1, out_b)
            issue_out(c1, out_b, sem_b)
            return carry

        lax.fori_loop(0, NCHUNK // 2, pair_body, 0)
        wait_out(NCHUNK - 2, out_a, sem_a)
        wait_out(NCHUNK - 1, out_b, sem_b)

    @pl.when(cid == 0)
    def _cont_side():
        _build_tab(INV17, True)

        def compute_chunk(c, out_v):
            off = base + c * CHUNK
            pltpu.sync_copy(pc_hbm.at[pl.ds(off, CHUNK)], b_p)
            pltpu.sync_copy(dc_hbm.at[pl.ds(off, CHUNK)], b_d)
            pltpu.sync_copy(vals_hbm.at[pl.ds(off, CHUNK)], b_vf)

            @plsc.parallel_loop(0, MB, step=1, unroll=4)
            def mb_body(m):
                sl = pl.ds(m * 16, 16)
                db = b_d[sl] * 17
                pb = b_p[sl] * 17
                for k in range(16):
                    pe = plsc.load_gather(tab_v, [db + k])
                    pr = plsc.load_gather(pair_p, [pb + k])
                    out_v[pl.ds(k * CSTRIDE + m * 16, 16)] = pe + pr
                pe16 = plsc.load_gather(tab_v, [db + 16])
                out_v[pl.ds(16 * CSTRIDE + m * 16, 16)] = pe16 + b_vf[sl]

        _run_chunks(compute_chunk, outc_hbm, 17, N_CONT)

    @pl.when(cid == 1)
    def _categ_side():
        pltpu.sync_copy(valtab_hbm, valtab_v)

        def build_comb(i, c0):
            pr = pair_v[pl.ds((i >> 4) * 16, 16)]
            comb_v[pl.ds(i * 17, 16)] = pr + valtab_v[pl.ds(i * 16, 16)]
            return c0

        lax.fori_loop(0, P * V, build_comb, 0)

        _build_tab(INV16, False)

        def compute_chunk(c, out_v):
            off = base + c * CHUNK
            pltpu.sync_copy(pg_hbm.at[pl.ds(off, CHUNK)], b_p)
            pltpu.sync_copy(dg_hbm.at[pl.ds(off, CHUNK)], b_d)
            pltpu.sync_copy(vg_hbm.at[pl.ds(off, CHUNK)], b_vi)

            @plsc.parallel_loop(0, MB, step=1, unroll=4)
            def mb_body(m):
                sl = pl.ds(m * 16, 16)
                db = b_d[sl] * 17
                cb = (b_p[sl] * 16 + b_vi[sl]) * 17
                for k in range(16):
                    pe = plsc.load_gather(tab_v, [db + k])
                    cm = plsc.load_gather(comb_v, [cb + k])
                    out_v[pl.ds(k * CSTRIDE + m * 16, 16)] = pe + cm

        _run_chunks(compute_chunk, outg_hbm, 16, N_CATEG)


@jax.jit
def kernel(ent_attr_ids_cont, vals_cont, days_cont,
           ent_attr_ids_categ, vocab_ids_categ, days_categ,
           pair_emb, categ_val_emb):
    mesh = plsc.VectorSubcoreMesh(core_axis_name="c", subcore_axis_name="s")
    f = pl.kernel(
        _embed_body,
        out_type=(jax.ShapeDtypeStruct((17 * N_CONT,), jnp.float32),
                  jax.ShapeDtypeStruct((16 * N_CATEG,), jnp.float32)),
        mesh=mesh,
        compiler_params=pltpu.CompilerParams(needs_layout_passes=False),
        scratch_types=[
            pltpu.VMEM((NDAYS * 17,), jnp.float32),
            pltpu.VMEM((P * D,), jnp.float32),
            pltpu.VMEM((P * 17,), jnp.float32),
            pltpu.VMEM((P * V * D,), jnp.float32),
            pltpu.VMEM((P * V * 17,), jnp.float32),
            pltpu.VMEM((CHUNK,), jnp.int32),
            pltpu.VMEM((CHUNK,), jnp.int32),
            pltpu.VMEM((CHUNK,), jnp.int32),
            pltpu.VMEM((CHUNK,), jnp.float32),
            pltpu.VMEM((17 * CSTRIDE,), jnp.float32),
            pltpu.VMEM((17 * CSTRIDE,), jnp.float32),
            pltpu.SemaphoreType.DMA,
            pltpu.SemaphoreType.DMA,
        ],
    )
    outc, outg = f(ent_attr_ids_cont.astype(jnp.int32), vals_cont,
                   days_cont.astype(jnp.int32),
                   ent_attr_ids_categ.astype(jnp.int32),
                   vocab_ids_categ.astype(jnp.int32),
                   days_categ.astype(jnp.int32),
                   pair_emb.reshape(P * D), categ_val_emb.reshape(P * V * D))
    return outc.reshape(17, N_CONT).T, outg.reshape(16, N_CATEG).T


# speedup vs baseline: 1.2823x; 1.0244x over previous
import functools

import jax
import jax.numpy as jnp
from jax import lax
from jax.experimental import pallas as pl
from jax.experimental.pallas import tpu as pltpu
from jax.experimental.pallas import tpu_sc as plsc

P = 16
V = 16
D = 16
N_CONT = 262144
N_CATEG = 262144
NDAYS = 3664

NC = 2
NS = 16
ROWS_T = N_CONT // NS
CHUNK = 1024
NCHUNK = ROWS_T // CHUNK
MB = CHUNK // 16
CSTRIDE = CHUNK + 8

TWOPI = 6.283185307179586
INV2PI = 1.0 / TWOPI

SIN_C = (0.9999791148943297, -0.1666240153829831, 0.00830884993122673,
         -0.00019263169952744158, 2.147049615625063e-06)
COS_C = (0.9999992107412203, -0.4999942131500665, 0.04165977758594538,
         -0.0013858789204833017, 2.4202932054760706e-05,
         -2.1972921876445284e-07)

INV17 = tuple(10000.0 ** (-(2 * j) / 17.0) for j in range(9))
INV16 = tuple(10000.0 ** (-(2 * j) / 16.0) for j in range(8))


def _range_reduce(ang):
    q = (ang * INV2PI + 0.5).astype(jnp.int32).astype(jnp.float32)
    return ang - q * TWOPI


def _sin_poly(r, r2):
    s = jnp.float32(SIN_C[-1])
    for c in SIN_C[-2::-1]:
        s = s * r2 + c
    return s * r


def _cos_poly(r2):
    c = jnp.float32(COS_C[-1])
    for cc in COS_C[-2::-1]:
        c = c * r2 + cc
    return c


def _embed_body(pc_hbm, vals_hbm, dc_hbm, pg_hbm, vg_hbm, dg_hbm,
                pair_hbm, valtab_hbm, outc_hbm, outg_hbm,
                tab_v, pair_v, pair_p, valtab_v, comb_v,
                b_p, b_d, b_vi, b_vf, out_a, out_b, sem_a, sem_b):
    cid = lax.axis_index("c")
    sid = lax.axis_index("s")
    base = sid * ROWS_T
    iota = lax.iota(jnp.int32, 16)

    pltpu.sync_copy(pair_hbm, pair_v)

    def repack(i, c0):
        pair_p[pl.ds(i * 17, 16)] = pair_v[pl.ds(i * 16, 16)]
        return c0

    lax.fori_loop(0, P, repack, 0)

    def _build_tab(inv, with_col16):
        @plsc.parallel_loop(0, NDAYS // 16, step=1, unroll=2)
        def build(t):
            d_f = (iota + t * 16).astype(jnp.float32)
            db = (iota + t * 16) * 17
            for j in range(8):
                r = _range_reduce(d_f * inv[j])
                r2 = r * r
                plsc.store_scatter(tab_v, [db + (2 * j)], _sin_poly(r, r2))
                plsc.store_scatter(tab_v, [db + (2 * j + 1)], _cos_poly(r2))
            if with_col16:
                r = _range_reduce(d_f * INV17[8])
                plsc.store_scatter(tab_v, [db + 16], _sin_poly(r, r * r))

    def _run_chunks(compute_chunk, out_hbm, ncols, n_total):
        def issue_out(c, buf, sem):
            off = base + c * CHUNK
            for k in range(ncols):
                pltpu.async_copy(buf.at[pl.ds(k * CSTRIDE, CHUNK)],
                                 out_hbm.at[pl.ds(k * n_total + off, CHUNK)],
                                 sem)

        def wait_out(c, buf, sem):
            off = base + jnp.maximum(c, 0) * CHUNK
            for k in range(ncols):
                pltpu.make_async_copy(
                    buf.at[pl.ds(k * CSTRIDE, CHUNK)],
                    out_hbm.at[pl.ds(k * n_total + off, CHUNK)],
                    sem).wait()

        def pair_body(cc, carry):
            c0 = cc * 2
            c1 = c0 + 1

            @pl.when(cc > 0)
            def _wa():
                wait_out(c0 - 2, out_a, sem_a)

            compute_chunk(c0, out_a)
            issue_out(c0, out_a, sem_a)

            @pl.when(cc > 0)
            def _wb():
                wait_out(c1 - 2, out_b, sem_b)

            compute_chunk(c1, out_b)
            issue_out(c1, out_b, sem_b)
            return carry

        lax.fori_loop(0, NCHUNK // 2, pair_body, 0)
        wait_out(NCHUNK - 2, out_a, sem_a)
        wait_out(NCHUNK - 1, out_b, sem_b)

    @pl.when(cid == 0)
    def _cont_side():
        _build_tab(INV17, True)

        def compute_chunk(c, out_v):
            off = base + c * CHUNK
            pltpu.sync_copy(pc_hbm.at[pl.ds(off, CHUNK)], b_p)
            pltpu.sync_copy(dc_hbm.at[pl.ds(off, CHUNK)], b_d)
            pltpu.sync_copy(vals_hbm.at[pl.ds(off, CHUNK)], b_vf)

            @plsc.parallel_loop(0, MB, step=1, unroll=8)
            def mb_body(m):
                sl = pl.ds(m * 16, 16)
                db = b_d[sl] * 17
                pb = b_p[sl] * 17
                for k in range(16):
                    pe = plsc.load_gather(tab_v, [db + k])
                    pr = plsc.load_gather(pair_p, [pb + k])
                    out_v[pl.ds(k * CSTRIDE + m * 16, 16)] = pe + pr
                pe16 = plsc.load_gather(tab_v, [db + 16])
                out_v[pl.ds(16 * CSTRIDE + m * 16, 16)] = pe16 + b_vf[sl]

        _run_chunks(compute_chunk, outc_hbm, 17, N_CONT)

    @pl.when(cid == 1)
    def _categ_side():
        pltpu.sync_copy(valtab_hbm, valtab_v)

        def build_comb(i, c0):
            pr = pair_v[pl.ds((i >> 4) * 16, 16)]
            comb_v[pl.ds(i * 17, 16)] = pr + valtab_v[pl.ds(i * 16, 16)]
            return c0

        lax.fori_loop(0, P * V, build_comb, 0)

        _build_tab(INV16, False)

        def compute_chunk(c, out_v):
            off = base + c * CHUNK
            pltpu.sync_copy(pg_hbm.at[pl.ds(off, CHUNK)], b_p)
            pltpu.sync_copy(dg_hbm.at[pl.ds(off, CHUNK)], b_d)
            pltpu.sync_copy(vg_hbm.at[pl.ds(off, CHUNK)], b_vi)

            @plsc.parallel_loop(0, MB, step=1, unroll=8)
            def mb_body(m):
                sl = pl.ds(m * 16, 16)
                db = b_d[sl] * 17
                cb = (b_p[sl] * 16 + b_vi[sl]) * 17
                for k in range(16):
                    pe = plsc.load_gather(tab_v, [db + k])
                    cm = plsc.load_gather(comb_v, [cb + k])
                    out_v[pl.ds(k * CSTRIDE + m * 16, 16)] = pe + cm

        _run_chunks(compute_chunk, outg_hbm, 16, N_CATEG)


@jax.jit
def kernel(ent_attr_ids_cont, vals_cont, days_cont,
           ent_attr_ids_categ, vocab_ids_categ, days_categ,
           pair_emb, categ_val_emb):
    mesh = plsc.VectorSubcoreMesh(core_axis_name="c", subcore_axis_name="s")
    f = pl.kernel(
        _embed_body,
        out_type=(jax.ShapeDtypeStruct((17 * N_CONT,), jnp.float32),
                  jax.ShapeDtypeStruct((16 * N_CATEG,), jnp.float32)),
        mesh=mesh,
        compiler_params=pltpu.CompilerParams(needs_layout_passes=False),
        scratch_types=[
            pltpu.VMEM((NDAYS * 17,), jnp.float32),
            pltpu.VMEM((P * D,), jnp.float32),
            pltpu.VMEM((P * 17,), jnp.float32),
            pltpu.VMEM((P * V * D,), jnp.float32),
            pltpu.VMEM((P * V * 17,), jnp.float32),
            pltpu.VMEM((CHUNK,), jnp.int32),
            pltpu.VMEM((CHUNK,), jnp.int32),
            pltpu.VMEM((CHUNK,), jnp.int32),
            pltpu.VMEM((CHUNK,), jnp.float32),
            pltpu.VMEM((17 * CSTRIDE,), jnp.float32),
            pltpu.VMEM((17 * CSTRIDE,), jnp.float32),
            pltpu.SemaphoreType.DMA,
            pltpu.SemaphoreType.DMA,
        ],
    )
    outc, outg = f(ent_attr_ids_cont.astype(jnp.int32), vals_cont,
                   days_cont.astype(jnp.int32),
                   ent_attr_ids_categ.astype(jnp.int32),
                   vocab_ids_categ.astype(jnp.int32),
                   days_categ.astype(jnp.int32),
                   pair_emb.reshape(P * D), categ_val_emb.reshape(P * V * D))
    return outc.reshape(17, N_CONT).T, outg.reshape(16, N_CATEG).T


# speedup vs baseline: 1.4334x; 1.1178x over previous
import functools

import jax
import jax.numpy as jnp
from jax import lax
from jax.experimental import pallas as pl
from jax.experimental.pallas import tpu as pltpu
from jax.experimental.pallas import tpu_sc as plsc

P = 16
V = 16
D = 16
N_CONT = 262144
N_CATEG = 262144
NDAYS = 3664

NC = 2
NS = 16
ROWS_T = N_CONT // NS
CHUNK = 1024
NCHUNK = ROWS_T // CHUNK
MB = CHUNK // 16
CSTRIDE = CHUNK + 8

TWOPI = 6.283185307179586
INV2PI = 1.0 / TWOPI

SIN_C = (0.9999791148943297, -0.1666240153829831, 0.00830884993122673,
         -0.00019263169952744158, 2.147049615625063e-06)
COS_C = (0.9999992107412203, -0.4999942131500665, 0.04165977758594538,
         -0.0013858789204833017, 2.4202932054760706e-05,
         -2.1972921876445284e-07)

INV17 = tuple(10000.0 ** (-(2 * j) / 17.0) for j in range(9))
INV16 = tuple(10000.0 ** (-(2 * j) / 16.0) for j in range(8))


def _range_reduce(ang):
    q = (ang * INV2PI + 0.5).astype(jnp.int32).astype(jnp.float32)
    return ang - q * TWOPI


def _sin_poly(r, r2):
    s = jnp.float32(SIN_C[-1])
    for c in SIN_C[-2::-1]:
        s = s * r2 + c
    return s * r


def _cos_poly(r2):
    c = jnp.float32(COS_C[-1])
    for cc in COS_C[-2::-1]:
        c = c * r2 + cc
    return c


def _embed_body(pc_hbm, vals_hbm, dc_hbm, pg_hbm, vg_hbm, dg_hbm,
                pair_hbm, valtab_hbm, outc_hbm, outg_hbm,
                tab_v, pair_v, pair_p, valtab_v, comb_v,
                b_p, b_d, b_vi, b_vf, out_a, out_b, sem_a, sem_b, sem_i):
    cid = lax.axis_index("c")
    sid = lax.axis_index("s")
    base = sid * ROWS_T
    iota = lax.iota(jnp.int32, 16)

    pltpu.sync_copy(pair_hbm, pair_v)

    def repack(i, c0):
        pair_p[pl.ds(i * 17, 16)] = pair_v[pl.ds(i * 16, 16)]
        return c0

    lax.fori_loop(0, P, repack, 0)

    def _build_tab(inv, with_col16):
        @plsc.parallel_loop(0, NDAYS // 16, step=1, unroll=2)
        def build(t):
            d_f = (iota + t * 16).astype(jnp.float32)
            db = (iota + t * 16) * 17
            for j in range(8):
                r = _range_reduce(d_f * inv[j])
                r2 = r * r
                plsc.store_scatter(tab_v, [db + (2 * j)], _sin_poly(r, r2))
                plsc.store_scatter(tab_v, [db + (2 * j + 1)], _cos_poly(r2))
            if with_col16:
                r = _range_reduce(d_f * INV17[8])
                plsc.store_scatter(tab_v, [db + 16], _sin_poly(r, r * r))

    def _run_chunks(compute_chunk, out_hbm, ncols, n_total):
        def issue_out(c, buf, sem):
            off = base + c * CHUNK
            for k in range(ncols):
                pltpu.async_copy(buf.at[pl.ds(k * CSTRIDE, CHUNK)],
                                 out_hbm.at[pl.ds(k * n_total + off, CHUNK)],
                                 sem)

        def wait_out(c, buf, sem):
            off = base + jnp.maximum(c, 0) * CHUNK
            for k in range(ncols):
                pltpu.make_async_copy(
                    buf.at[pl.ds(k * CSTRIDE, CHUNK)],
                    out_hbm.at[pl.ds(k * n_total + off, CHUNK)],
                    sem).wait()

        def pair_body(cc, carry):
            c0 = cc * 2
            c1 = c0 + 1

            @pl.when(cc > 0)
            def _wa():
                wait_out(c0 - 2, out_a, sem_a)

            compute_chunk(c0, out_a)
            issue_out(c0, out_a, sem_a)

            @pl.when(cc > 0)
            def _wb():
                wait_out(c1 - 2, out_b, sem_b)

            compute_chunk(c1, out_b)
            issue_out(c1, out_b, sem_b)
            return carry

        lax.fori_loop(0, NCHUNK // 2, pair_body, 0)
        wait_out(NCHUNK - 2, out_a, sem_a)
        wait_out(NCHUNK - 1, out_b, sem_b)

    @pl.when(cid == 0)
    def _cont_side():
        _build_tab(INV17, True)

        def compute_chunk(c, out_v):
            off = base + c * CHUNK
            h1 = pltpu.async_copy(pc_hbm.at[pl.ds(off, CHUNK)], b_p, sem_i)
            h2 = pltpu.async_copy(dc_hbm.at[pl.ds(off, CHUNK)], b_d, sem_i)
            h3 = pltpu.async_copy(vals_hbm.at[pl.ds(off, CHUNK)], b_vf, sem_i)
            h1.wait(); h2.wait(); h3.wait()

            @plsc.parallel_loop(0, MB, step=1, unroll=8)
            def mb_body(m):
                sl = pl.ds(m * 16, 16)
                db = b_d[sl] * 17
                pb = b_p[sl] * 17
                for k in range(16):
                    pe = plsc.load_gather(tab_v, [db + k])
                    pr = plsc.load_gather(pair_p, [pb + k])
                    out_v[pl.ds(k * CSTRIDE + m * 16, 16)] = pe + pr
                pe16 = plsc.load_gather(tab_v, [db + 16])
                out_v[pl.ds(16 * CSTRIDE + m * 16, 16)] = pe16 + b_vf[sl]

        _run_chunks(compute_chunk, outc_hbm, 17, N_CONT)

    @pl.when(cid == 1)
    def _categ_side():
        pltpu.sync_copy(valtab_hbm, valtab_v)

        def build_comb(i, c0):
            pr = pair_v[pl.ds((i >> 4) * 16, 16)]
            comb_v[pl.ds(i * 17, 16)] = pr + valtab_v[pl.ds(i * 16, 16)]
            return c0

        lax.fori_loop(0, P * V, build_comb, 0)

        _build_tab(INV16, False)

        def compute_chunk(c, out_v):
            off = base + c * CHUNK
            h1 = pltpu.async_copy(pg_hbm.at[pl.ds(off, CHUNK)], b_p, sem_i)
            h2 = pltpu.async_copy(dg_hbm.at[pl.ds(off, CHUNK)], b_d, sem_i)
            h3 = pltpu.async_copy(vg_hbm.at[pl.ds(off, CHUNK)], b_vi, sem_i)
            h1.wait(); h2.wait(); h3.wait()

            @plsc.parallel_loop(0, MB, step=1, unroll=8)
            def mb_body(m):
                sl = pl.ds(m * 16, 16)
                db = b_d[sl] * 17
                cb = (b_p[sl] * 16 + b_vi[sl]) * 17
                for k in range(16):
                    pe = plsc.load_gather(tab_v, [db + k])
                    cm = plsc.load_gather(comb_v, [cb + k])
                    out_v[pl.ds(k * CSTRIDE + m * 16, 16)] = pe + cm

        _run_chunks(compute_chunk, outg_hbm, 16, N_CATEG)


@jax.jit
def kernel(ent_attr_ids_cont, vals_cont, days_cont,
           ent_attr_ids_categ, vocab_ids_categ, days_categ,
           pair_emb, categ_val_emb):
    mesh = plsc.VectorSubcoreMesh(core_axis_name="c", subcore_axis_name="s")
    f = pl.kernel(
        _embed_body,
        out_type=(jax.ShapeDtypeStruct((17 * N_CONT,), jnp.float32),
                  jax.ShapeDtypeStruct((16 * N_CATEG,), jnp.float32)),
        mesh=mesh,
        compiler_params=pltpu.CompilerParams(needs_layout_passes=False),
        scratch_types=[
            pltpu.VMEM((NDAYS * 17,), jnp.float32),
            pltpu.VMEM((P * D,), jnp.float32),
            pltpu.VMEM((P * 17,), jnp.float32),
            pltpu.VMEM((P * V * D,), jnp.float32),
            pltpu.VMEM((P * V * 17,), jnp.float32),
            pltpu.VMEM((CHUNK,), jnp.int32),
            pltpu.VMEM((CHUNK,), jnp.int32),
            pltpu.VMEM((CHUNK,), jnp.int32),
            pltpu.VMEM((CHUNK,), jnp.float32),
            pltpu.VMEM((17 * CSTRIDE,), jnp.float32),
            pltpu.VMEM((17 * CSTRIDE,), jnp.float32),
            pltpu.SemaphoreType.DMA,
            pltpu.SemaphoreType.DMA,
            pltpu.SemaphoreType.DMA,
        ],
    )
    outc, outg = f(ent_attr_ids_cont.astype(jnp.int32), vals_cont,
                   days_cont.astype(jnp.int32),
                   ent_attr_ids_categ.astype(jnp.int32),
                   vocab_ids_categ.astype(jnp.int32),
                   days_categ.astype(jnp.int32),
                   pair_emb.reshape(P * D), categ_val_emb.reshape(P * V * D))
    return outc.reshape(17, N_CONT).T, outg.reshape(16, N_CATEG).T


# speedup vs baseline: 1.5724x; 1.0969x over previous
import functools

import jax
import jax.numpy as jnp
from jax import lax
from jax.experimental import pallas as pl
from jax.experimental.pallas import tpu as pltpu
from jax.experimental.pallas import tpu_sc as plsc

P = 16
V = 16
D = 16
N_CONT = 262144
N_CATEG = 262144
NDAYS = 3664

NC = 2
NS = 16
ROWS_T = N_CONT // NS
CHUNK = 1024
NCHUNK = ROWS_T // CHUNK
MB = CHUNK // 16
CSTRIDE = CHUNK + 8

TWOPI = 6.283185307179586
INV2PI = 1.0 / TWOPI

SIN_C = (0.9999791148943297, -0.1666240153829831, 0.00830884993122673,
         -0.00019263169952744158, 2.147049615625063e-06)
COS_C = (0.9999992107412203, -0.4999942131500665, 0.04165977758594538,
         -0.0013858789204833017, 2.4202932054760706e-05,
         -2.1972921876445284e-07)

INV17 = tuple(10000.0 ** (-(2 * j) / 17.0) for j in range(9))
INV16 = tuple(10000.0 ** (-(2 * j) / 16.0) for j in range(8))


def _range_reduce(ang):
    q = (ang * INV2PI + 0.5).astype(jnp.int32).astype(jnp.float32)
    return ang - q * TWOPI


def _sin_poly(r, r2):
    s = jnp.float32(SIN_C[-1])
    for c in SIN_C[-2::-1]:
        s = s * r2 + c
    return s * r


def _cos_poly(r2):
    c = jnp.float32(COS_C[-1])
    for cc in COS_C[-2::-1]:
        c = c * r2 + cc
    return c


def _embed_body(pc_hbm, vals_hbm, dc_hbm, pg_hbm, vg_hbm, dg_hbm,
                pair_hbm, valtab_hbm, outc_hbm, outg_hbm,
                tab_v, pair_v, pair_p, valtab_v, comb_v,
                b_pa, b_da, b_xa, b_pb, b_db, b_xb, b_vfa, b_vfb,
                out_a, out_b, sem_a, sem_b, sem_ia, sem_ib):
    cid = lax.axis_index("c")
    sid = lax.axis_index("s")
    base = sid * ROWS_T
    iota = lax.iota(jnp.int32, 16)

    pltpu.sync_copy(pair_hbm, pair_v)

    def repack(i, c0):
        pair_p[pl.ds(i * 17, 16)] = pair_v[pl.ds(i * 16, 16)]
        return c0

    lax.fori_loop(0, P, repack, 0)

    def _build_tab(inv, with_col16):
        @plsc.parallel_loop(0, NDAYS // 16, step=1, unroll=2)
        def build(t):
            d_f = (iota + t * 16).astype(jnp.float32)
            db = (iota + t * 16) * 17
            for j in range(8):
                r = _range_reduce(d_f * inv[j])
                r2 = r * r
                plsc.store_scatter(tab_v, [db + (2 * j)], _sin_poly(r, r2))
                plsc.store_scatter(tab_v, [db + (2 * j + 1)], _cos_poly(r2))
            if with_col16:
                r = _range_reduce(d_f * INV17[8])
                plsc.store_scatter(tab_v, [db + 16], _sin_poly(r, r * r))

    def _run_chunks(in_copies, compute_chunk, out_hbm, ncols, n_total):
        def issue_in(c, ins, sem):
            for s, d in in_copies(c, ins):
                pltpu.async_copy(s, d, sem)

        def wait_in(c, ins, sem):
            for s, d in in_copies(c, ins):
                pltpu.make_async_copy(s, d, sem).wait()

        def issue_out(c, buf, sem):
            off = base + c * CHUNK
            for k in range(ncols):
                pltpu.async_copy(buf.at[pl.ds(k * CSTRIDE, CHUNK)],
                                 out_hbm.at[pl.ds(k * n_total + off, CHUNK)],
                                 sem)

        def wait_out(c, buf, sem):
            off = base + jnp.maximum(c, 0) * CHUNK
            for k in range(ncols):
                pltpu.make_async_copy(
                    buf.at[pl.ds(k * CSTRIDE, CHUNK)],
                    out_hbm.at[pl.ds(k * n_total + off, CHUNK)],
                    sem).wait()

        ins_a = (b_pa, b_da, b_xa)
        ins_b = (b_pb, b_db, b_xb)
        issue_in(0, ins_a, sem_ia)

        def pair_body(cc, carry):
            c0 = cc * 2
            c1 = c0 + 1

            wait_in(c0, ins_a, sem_ia)
            issue_in(c1, ins_b, sem_ib)

            @pl.when(cc > 0)
            def _wa():
                wait_out(c0 - 2, out_a, sem_a)

            compute_chunk(c0, out_a, ins_a)
            issue_out(c0, out_a, sem_a)

            wait_in(c1, ins_b, sem_ib)

            @pl.when(cc < NCHUNK // 2 - 1)
            def _ia():
                issue_in(c0 + 2, ins_a, sem_ia)

            @pl.when(cc > 0)
            def _wb():
                wait_out(c1 - 2, out_b, sem_b)

            compute_chunk(c1, out_b, ins_b)
            issue_out(c1, out_b, sem_b)
            return carry

        lax.fori_loop(0, NCHUNK // 2, pair_body, 0)
        wait_out(NCHUNK - 2, out_a, sem_a)
        wait_out(NCHUNK - 1, out_b, sem_b)

    @pl.when(cid == 0)
    def _cont_side():
        _build_tab(INV17, True)

        def in_copies(c, ins):
            off = base + jnp.minimum(c, NCHUNK - 1) * CHUNK
            vf = b_vfa if ins[0] is b_pa else b_vfb
            return [(pc_hbm.at[pl.ds(off, CHUNK)], ins[0]),
                    (dc_hbm.at[pl.ds(off, CHUNK)], ins[1]),
                    (vals_hbm.at[pl.ds(off, CHUNK)], vf)]

        def compute_chunk(c, out_v, ins):
            b_p, b_d, _ = ins
            b_vf = b_vfa if ins[0] is b_pa else b_vfb

            @plsc.parallel_loop(0, MB, step=1, unroll=8)
            def mb_body(m):
                sl = pl.ds(m * 16, 16)
                db = b_d[sl] * 17
                pb = b_p[sl] * 17
                for k in range(16):
                    pe = plsc.load_gather(tab_v, [db + k])
                    pr = plsc.load_gather(pair_p, [pb + k])
                    out_v[pl.ds(k * CSTRIDE + m * 16, 16)] = pe + pr
                pe16 = plsc.load_gather(tab_v, [db + 16])
                out_v[pl.ds(16 * CSTRIDE + m * 16, 16)] = pe16 + b_vf[sl]

        _run_chunks(in_copies, compute_chunk, outc_hbm, 17, N_CONT)

    @pl.when(cid == 1)
    def _categ_side():
        pltpu.sync_copy(valtab_hbm, valtab_v)

        def build_comb(i, c0):
            pr = pair_v[pl.ds((i >> 4) * 16, 16)]
            comb_v[pl.ds(i * 17, 16)] = pr + valtab_v[pl.ds(i * 16, 16)]
            return c0

        lax.fori_loop(0, P * V, build_comb, 0)

        _build_tab(INV16, False)

        def in_copies(c, ins):
            off = base + jnp.minimum(c, NCHUNK - 1) * CHUNK
            return [(pg_hbm.at[pl.ds(off, CHUNK)], ins[0]),
                    (dg_hbm.at[pl.ds(off, CHUNK)], ins[1]),
                    (vg_hbm.at[pl.ds(off, CHUNK)], ins[2])]

        def compute_chunk(c, out_v, ins):
            b_p, b_d, b_vi = ins

            @plsc.parallel_loop(0, MB, step=1, unroll=8)
            def mb_body(m):
                sl = pl.ds(m * 16, 16)
                db = b_d[sl] * 17
                cb = (b_p[sl] * 16 + b_vi[sl]) * 17
                for k in range(16):
                    pe = plsc.load_gather(tab_v, [db + k])
                    cm = plsc.load_gather(comb_v, [cb + k])
                    out_v[pl.ds(k * CSTRIDE + m * 16, 16)] = pe + cm

        _run_chunks(in_copies, compute_chunk, outg_hbm, 16, N_CATEG)


@jax.jit
def kernel(ent_attr_ids_cont, vals_cont, days_cont,
           ent_attr_ids_categ, vocab_ids_categ, days_categ,
           pair_emb, categ_val_emb):
    mesh = plsc.VectorSubcoreMesh(core_axis_name="c", subcore_axis_name="s")
    f = pl.kernel(
        _embed_body,
        out_type=(jax.ShapeDtypeStruct((17 * N_CONT,), jnp.float32),
                  jax.ShapeDtypeStruct((16 * N_CATEG,), jnp.float32)),
        mesh=mesh,
        compiler_params=pltpu.CompilerParams(needs_layout_passes=False),
        scratch_types=[
            pltpu.VMEM((NDAYS * 17,), jnp.float32),
            pltpu.VMEM((P * D,), jnp.float32),
            pltpu.VMEM((P * 17,), jnp.float32),
            pltpu.VMEM((P * V * D,), jnp.float32),
            pltpu.VMEM((P * V * 17,), jnp.float32),
            pltpu.VMEM((CHUNK,), jnp.int32),
            pltpu.VMEM((CHUNK,), jnp.int32),
            pltpu.VMEM((CHUNK,), jnp.int32),
            pltpu.VMEM((CHUNK,), jnp.int32),
            pltpu.VMEM((CHUNK,), jnp.int32),
            pltpu.VMEM((CHUNK,), jnp.int32),
            pltpu.VMEM((CHUNK,), jnp.float32),
            pltpu.VMEM((CHUNK,), jnp.float32),
            pltpu.VMEM((17 * CSTRIDE,), jnp.float32),
            pltpu.VMEM((17 * CSTRIDE,), jnp.float32),
            pltpu.SemaphoreType.DMA,
            pltpu.SemaphoreType.DMA,
            pltpu.SemaphoreType.DMA,
            pltpu.SemaphoreType.DMA,
        ],
    )
    outc, outg = f(ent_attr_ids_cont.astype(jnp.int32), vals_cont,
                   days_cont.astype(jnp.int32),
                   ent_attr_ids_categ.astype(jnp.int32),
                   vocab_ids_categ.astype(jnp.int32),
                   days_categ.astype(jnp.int32),
                   pair_emb.reshape(P * D), categ_val_emb.reshape(P * V * D))
    return outc.reshape(17, N_CONT).T, outg.reshape(16, N_CATEG).T
